# baseline (speedup 1.0000x reference)
import functools

import jax
import jax.numpy as jnp
from jax.experimental import pallas as pl

F32 = jnp.float32

NPTS = 128
KS = 32
KSC = 4
FD = 128
QT = 256
_KNOCK = 1e30


def _rup(x, m):
    return (x + m - 1) // m * m


def _dot(a, b):
    return jnp.dot(a, b, preferred_element_type=F32)


def _extract_min(dist):
    m = jnp.min(dist, axis=1, keepdims=True)
    selb = dist == m
    return selb.astype(F32), jnp.where(selb, _KNOCK, dist)


def _select_sums(sel, pt):
    cols = [jnp.sum(sel * pt[c:c + 1, :], axis=1, keepdims=True)
            for c in range(3)]
    return jnp.concatenate(cols, axis=1)


def _stage1_kernel(s_ref, noisy_t_ref,
                   wf1_ref, bf1_ref, wf2_ref, bf2_ref, wf3_ref, bf3_ref,
                   ws1x_ref, ws1z_ref, bs1_ref, ws2_ref, bs2_ref,
                   ws3_ref, bs3_ref, f_ref, estim_ref):
    s = s_ref[0]
    pt = noisy_t_ref[0]
    pp = jnp.sum(pt * pt, axis=0, keepdims=True)
    qq = jnp.sum(s * s, axis=1, keepdims=True)
    dist0 = qq + pp - 2.0 * _dot(s, pt)

    def body(k, dist):
        sel, dist = _extract_min(dist)
        cnt = jnp.sum(sel, axis=1, keepdims=True)
        f_ref[0, pl.ds(k * NPTS, NPTS), :] = _select_sums(sel, pt) / cnt
        return dist

    jax.lax.fori_loop(0, KS, body, dist0)
    f = f_ref[0]

    h = jnp.maximum(_dot(s, wf1_ref[...]) + bf1_ref[...], 0.0)
    h = jnp.maximum(_dot(h, wf2_ref[...]) + bf2_ref[...], 0.0)
    z = _dot(h, wf3_ref[...]) + bf3_ref[...]

    x2 = f - jnp.tile(s, (KS, 1))
    hx = _dot(x2, ws1x_ref[...]).reshape(KS, NPTS, FD)
    hz = _dot(z, ws1z_ref[...])
    h1 = jnp.maximum(hx + hz[None] + bs1_ref[...][None], 0.0)
    h1 = h1.reshape(KS * NPTS, FD)
    h2 = jnp.maximum(_dot(h1, ws2_ref[...]) + bs2_ref[...], 0.0)
    estim_ref[0] = _dot(h2, ws3_ref[...]) + bs3_ref[...]


def _kth_smallest(dist, k):
    qun, pm = dist.shape
    c = 128
    m = [jnp.full((qun, c), _KNOCK, F32) for _ in range(k)]
    for j in range(pm // c):
        t = dist[:, j * c:(j + 1) * c]
        for i in range(k):
            lo = jnp.minimum(m[i], t)
            t = jnp.maximum(m[i], t)
            m[i] = lo
    width = c
    while width > 1:
        half = width // 2
        a = [mi[:, :half] for mi in m]
        b = [mi[:, half:width] for mi in m]
        lo = [jnp.minimum(a[i], b[k - 1 - i]) for i in range(k)]
        s0, s2 = jnp.minimum(lo[0], lo[2]), jnp.maximum(lo[0], lo[2])
        s1, s3 = jnp.minimum(lo[1], lo[3]), jnp.maximum(lo[1], lo[3])
        m = [jnp.minimum(s0, s1), jnp.maximum(s0, s1),
             jnp.minimum(s2, s3), jnp.maximum(s2, s3)]
        width = half
    return m[k - 1]


def _stage2_kernel(q_ref, est_ref, clean_t4_ref, out_ref, *, scale):
    q = q_ref[0]
    pt4 = clean_t4_ref[0]
    pt = pt4[:3]
    pp = jnp.sum(pt * pt, axis=0, keepdims=True)
    qq = jnp.sum(q * q, axis=1, keepdims=True)
    dist = qq + pp - 2.0 * _dot(q, pt)
    t4 = _kth_smallest(dist, KSC)
    w = (dist <= t4).astype(F32)
    sums = jax.lax.dot_general(w, pt4, (((1,), (1,)), ((), ())),
                               preferred_element_type=F32,
                               precision=jax.lax.Precision.HIGHEST)
    ground = sums[:, :3] / sums[:, 3:4] - q
    diff = est_ref[0] - ground
    part = jnp.sum(diff * diff) * scale

    @pl.when((pl.program_id(0) == 0) & (pl.program_id(1) == 0))
    def _init():
        out_ref[...] = jnp.zeros_like(out_ref)

    out_ref[...] += part.reshape(1, 1)


def kernel(noisy_pc, clean_pc, Wf1, bf1, Wf2, bf2, Wf3, bf3,
           Ws1, bs1, Ws2, bs2, Ws3, bs3):
    B, N, _ = noisy_pc.shape
    M = clean_pc.shape[1]
    PN, PM = _rup(N, 128), _rup(M, 128)
    PADV = 1e5

    noisy_t = jnp.pad(jnp.transpose(noisy_pc, (0, 2, 1)),
                      ((0, 0), (0, 0), (0, PN - N)), constant_values=PADV)
    clean_t = jnp.pad(jnp.transpose(clean_pc, (0, 2, 1)),
                      ((0, 0), (0, 0), (0, PM - M)), constant_values=PADV)
    clean_t4 = jnp.concatenate(
        [clean_t, jnp.ones((B, 1, PM), F32)], axis=1)
    s = noisy_pc[:, :NPTS, :]

    b2 = lambda d: jnp.reshape(d, (1, -1))
    full = lambda shp: pl.BlockSpec(shp, lambda b: (0,) * len(shp))

    f, est = pl.pallas_call(
        _stage1_kernel,
        grid=(B,),
        in_specs=[
            pl.BlockSpec((1, NPTS, 3), lambda b: (b, 0, 0)),
            pl.BlockSpec((1, 3, PN), lambda b: (b, 0, 0)),
            full((3, 64)), full((1, 64)),
            full((64, 128)), full((1, 128)),
            full((128, FD)), full((1, FD)),
            full((3, FD)), full((FD, FD)), full((1, FD)),
            full((FD, FD)), full((1, FD)),
            full((FD, 3)), full((1, 3)),
        ],
        out_specs=[
            pl.BlockSpec((1, KS * NPTS, 3), lambda b: (b, 0, 0)),
            pl.BlockSpec((1, KS * NPTS, 3), lambda b: (b, 0, 0)),
        ],
        out_shape=[
            jax.ShapeDtypeStruct((B, KS * NPTS, 3), F32),
            jax.ShapeDtypeStruct((B, KS * NPTS, 3), F32),
        ],
    )(s, noisy_t,
      Wf1, b2(bf1), Wf2, b2(bf2), Wf3, b2(bf3),
      Ws1[:3], Ws1[3:], b2(bs1), Ws2, b2(bs2), Ws3, b2(bs3))

    rows = B * KS * NPTS
    loss = pl.pallas_call(
        functools.partial(_stage2_kernel, scale=50.0 / rows),
        grid=(B, (KS * NPTS) // QT),
        in_specs=[
            pl.BlockSpec((1, QT, 3), lambda b, t: (b, t, 0)),
            pl.BlockSpec((1, QT, 3), lambda b, t: (b, t, 0)),
            pl.BlockSpec((1, 4, PM), lambda b, t: (b, 0, 0)),
        ],
        out_specs=pl.BlockSpec((1, 1), lambda b, t: (0, 0)),
        out_shape=jax.ShapeDtypeStruct((1, 1), F32),
    )(f, est, clean_t4)
    return loss[0, 0]


# speedup vs baseline: 2.2961x; 2.2961x over previous
import functools

import jax
import jax.numpy as jnp
from jax.experimental import pallas as pl

F32 = jnp.float32

NPTS = 128
KS = 32
KSC = 4
FD = 128
QT = 256
_DEPTH = 8
_KNOCK = 1e30


def _rup(x, m):
    return (x + m - 1) // m * m


def _dot(a, b):
    return jnp.dot(a, b, preferred_element_type=F32)


def _extract_min(dist):
    m = jnp.min(dist, axis=1, keepdims=True)
    selb = dist == m
    return selb.astype(F32), jnp.where(selb, _KNOCK, dist)


def _select_sums(sel, pt):
    cols = [jnp.sum(sel * pt[c:c + 1, :], axis=1, keepdims=True)
            for c in range(3)]
    return jnp.concatenate(cols, axis=1)


def _stage1_kernel(s_ref, noisy_t_ref,
                   wf1_ref, bf1_ref, wf2_ref, bf2_ref, wf3_ref, bf3_ref,
                   ws1x_ref, ws1z_ref, bs1_ref, ws2_ref, bs2_ref,
                   ws3_ref, bs3_ref, f_ref, estim_ref):
    s = s_ref[0]
    pt = noisy_t_ref[0]
    pp = jnp.sum(pt * pt, axis=0, keepdims=True)
    qq = jnp.sum(s * s, axis=1, keepdims=True)
    dist0 = qq + pp - 2.0 * _dot(s, pt)
    pn = dist0.shape[1]
    c = 128

    mv = [jnp.full((NPTS, c), _KNOCK, F32) for _ in range(_DEPTH)]
    mx = [[jnp.zeros((NPTS, c), F32) for _ in range(_DEPTH)]
          for _ in range(3)]
    for j in range(pn // c):
        t = dist0[:, j * c:(j + 1) * c]
        tc = [pt[d:d + 1, j * c:(j + 1) * c] for d in range(3)]
        for i in range(_DEPTH):
            take = t < mv[i]
            lo = jnp.minimum(mv[i], t)
            t = jnp.maximum(mv[i], t)
            mv[i] = lo
            for d in range(3):
                keep = jnp.where(take, tc[d], mx[d][i])
                tc[d] = jnp.where(take, mx[d][i], tc[d])
                mx[d][i] = keep

    pop_total = jnp.zeros((NPTS, 1), F32)
    m = None
    for k in range(KS):
        m = jnp.min(mv[0], axis=1, keepdims=True)
        selb = mv[0] == m
        sel = selb.astype(F32)
        cnt = jnp.sum(sel, axis=1, keepdims=True)
        cols = [jnp.sum(sel * mx[d][0], axis=1, keepdims=True)
                for d in range(3)]
        f_ref[0, pl.ds(k * NPTS, NPTS), :] = jnp.concatenate(cols, 1) / cnt
        pop_total = pop_total + cnt
        for i in range(_DEPTH - 1):
            mv[i] = jnp.where(selb, mv[i + 1], mv[i])
            for d in range(3):
                mx[d][i] = jnp.where(selb, mx[d][i + 1], mx[d][i])
        mv[_DEPTH - 1] = jnp.where(selb, _KNOCK, mv[_DEPTH - 1])

    n_le = jnp.sum((dist0 <= m).astype(F32), axis=1, keepdims=True)
    ok = jnp.max(jnp.abs(n_le - pop_total)) == 0.0

    @pl.when(jnp.logical_not(ok))
    def _exact_fallback():
        def body(k, dist):
            sel, dist = _extract_min(dist)
            cnt = jnp.sum(sel, axis=1, keepdims=True)
            f_ref[0, pl.ds(k * NPTS, NPTS), :] = _select_sums(sel, pt) / cnt
            return dist

        jax.lax.fori_loop(0, KS, body, dist0)

    f = f_ref[0]

    h = jnp.maximum(_dot(s, wf1_ref[...]) + bf1_ref[...], 0.0)
    h = jnp.maximum(_dot(h, wf2_ref[...]) + bf2_ref[...], 0.0)
    z = _dot(h, wf3_ref[...]) + bf3_ref[...]

    x2 = f - jnp.tile(s, (KS, 1))
    hx = _dot(x2, ws1x_ref[...]).reshape(KS, NPTS, FD)
    hz = _dot(z, ws1z_ref[...])
    h1 = jnp.maximum(hx + hz[None] + bs1_ref[...][None], 0.0)
    h1 = h1.reshape(KS * NPTS, FD)
    h2 = jnp.maximum(_dot(h1, ws2_ref[...]) + bs2_ref[...], 0.0)
    estim_ref[0] = _dot(h2, ws3_ref[...]) + bs3_ref[...]


def _kth_smallest(dist, k):
    qun, pm = dist.shape
    c = 128
    m = [jnp.full((qun, c), _KNOCK, F32) for _ in range(k)]
    for j in range(pm // c):
        t = dist[:, j * c:(j + 1) * c]
        for i in range(k):
            lo = jnp.minimum(m[i], t)
            t = jnp.maximum(m[i], t)
            m[i] = lo
    width = c
    while width > 1:
        half = width // 2
        a = [mi[:, :half] for mi in m]
        b = [mi[:, half:width] for mi in m]
        lo = [jnp.minimum(a[i], b[k - 1 - i]) for i in range(k)]
        s0, s2 = jnp.minimum(lo[0], lo[2]), jnp.maximum(lo[0], lo[2])
        s1, s3 = jnp.minimum(lo[1], lo[3]), jnp.maximum(lo[1], lo[3])
        m = [jnp.minimum(s0, s1), jnp.maximum(s0, s1),
             jnp.minimum(s2, s3), jnp.maximum(s2, s3)]
        width = half
    return m[k - 1]


def _stage2_kernel(q_ref, est_ref, clean_t_ref, out_ref, *, scale):
    q = q_ref[0]
    pt = clean_t_ref[0]
    pp = jnp.sum(pt * pt, axis=0, keepdims=True)
    qq = jnp.sum(q * q, axis=1, keepdims=True)
    dist = qq + pp - 2.0 * _dot(q, pt)
    t4 = _kth_smallest(dist, KSC)
    w = (dist <= t4).astype(F32)
    cnt = jnp.sum(w, axis=1, keepdims=True)
    ground = _select_sums(w, pt) / cnt - q
    diff = est_ref[0] - ground
    part = jnp.sum(diff * diff) * scale

    @pl.when((pl.program_id(0) == 0) & (pl.program_id(1) == 0))
    def _init():
        out_ref[...] = jnp.zeros_like(out_ref)

    out_ref[...] += part.reshape(1, 1)


def kernel(noisy_pc, clean_pc, Wf1, bf1, Wf2, bf2, Wf3, bf3,
           Ws1, bs1, Ws2, bs2, Ws3, bs3):
    B, N, _ = noisy_pc.shape
    M = clean_pc.shape[1]
    PN, PM = _rup(N, 128), _rup(M, 128)
    PADV = 1e5

    noisy_t = jnp.pad(jnp.transpose(noisy_pc, (0, 2, 1)),
                      ((0, 0), (0, 0), (0, PN - N)), constant_values=PADV)
    clean_t = jnp.pad(jnp.transpose(clean_pc, (0, 2, 1)),
                      ((0, 0), (0, 0), (0, PM - M)), constant_values=PADV)
    s = noisy_pc[:, :NPTS, :]

    b2 = lambda d: jnp.reshape(d, (1, -1))
    full = lambda shp: pl.BlockSpec(shp, lambda b: (0,) * len(shp))

    f, est = pl.pallas_call(
        _stage1_kernel,
        grid=(B,),
        in_specs=[
            pl.BlockSpec((1, NPTS, 3), lambda b: (b, 0, 0)),
            pl.BlockSpec((1, 3, PN), lambda b: (b, 0, 0)),
            full((3, 64)), full((1, 64)),
            full((64, 128)), full((1, 128)),
            full((128, FD)), full((1, FD)),
            full((3, FD)), full((FD, FD)), full((1, FD)),
            full((FD, FD)), full((1, FD)),
            full((FD, 3)), full((1, 3)),
        ],
        out_specs=[
            pl.BlockSpec((1, KS * NPTS, 3), lambda b: (b, 0, 0)),
            pl.BlockSpec((1, KS * NPTS, 3), lambda b: (b, 0, 0)),
        ],
        out_shape=[
            jax.ShapeDtypeStruct((B, KS * NPTS, 3), F32),
            jax.ShapeDtypeStruct((B, KS * NPTS, 3), F32),
        ],
    )(s, noisy_t,
      Wf1, b2(bf1), Wf2, b2(bf2), Wf3, b2(bf3),
      Ws1[:3], Ws1[3:], b2(bs1), Ws2, b2(bs2), Ws3, b2(bs3))

    rows = B * KS * NPTS
    loss = pl.pallas_call(
        functools.partial(_stage2_kernel, scale=50.0 / rows),
        grid=(B, (KS * NPTS) // QT),
        in_specs=[
            pl.BlockSpec((1, QT, 3), lambda b, t: (b, t, 0)),
            pl.BlockSpec((1, QT, 3), lambda b, t: (b, t, 0)),
            pl.BlockSpec((1, 3, PM), lambda b, t: (b, 0, 0)),
        ],
        out_specs=pl.BlockSpec((1, 1), lambda b, t: (0, 0)),
        out_shape=jax.ShapeDtypeStruct((1, 1), F32),
    )(f, est, clean_t)
    return loss[0, 0]
